# baseline (speedup 1.0000x reference)
import functools

import jax
import jax.numpy as jnp
from jax import lax
from jax.experimental import pallas as pl
from jax.experimental.pallas import tpu as pltpu
from jax.experimental.pallas import tpu_sc as plsc

_NC = 2
_NS = 16
_NW = _NC * _NS
_CHUNK = 1000


def _sc_gather(row, col, user_emb, item_emb, base0, Es):
    D = user_emb.shape[1]
    per_w = Es // _NW
    iters = per_w // _CHUNK
    mesh = plsc.VectorSubcoreMesh(
        core_axis_name="c", subcore_axis_name="s",
        num_cores=_NC, num_subcores=_NS)

    assert iters % 2 == 1 and iters >= 5

    @functools.partial(
        pl.kernel,
        out_type=(jax.ShapeDtypeStruct((Es, D), jnp.float32),
                  jax.ShapeDtypeStruct((Es, D), jnp.float32)),
        mesh=mesh,
        scratch_types=[
            pltpu.VMEM((per_w,), jnp.int32),
            pltpu.VMEM((per_w,), jnp.int32),
            pltpu.VMEM((2, _CHUNK, D), jnp.float32),
            pltpu.VMEM((2, _CHUNK, D), jnp.float32),
            pltpu.SemaphoreType.DMA,
            [pltpu.SemaphoreType.DMA] * 2,
            [pltpu.SemaphoreType.DMA] * 2,
        ],
        compiler_params=pltpu.CompilerParams(use_tc_tiling_on_sc=False),
    )
    def k(row_hbm, col_hbm, uemb_hbm, iemb_hbm, ue_out, ie_out,
          ridx_v, cidx_v, ue_v, ie_v, sem_idx, sem_g, sem_w):
        wid = lax.axis_index("s") * _NC + lax.axis_index("c")
        wbase = pl.multiple_of(wid * per_w, 8)

        cp_r = pltpu.async_copy(
            row_hbm.at[pl.ds(base0 + wbase, per_w)], ridx_v, sem_idx)
        cp_c = pltpu.async_copy(
            col_hbm.at[pl.ds(base0 + wbase, per_w)], cidx_v, sem_idx)
        cp_r.wait()
        cp_c.wait()

        def start_gather(i, b):
            off = pl.multiple_of(i * _CHUNK, 8)
            pltpu.async_copy(
                uemb_hbm.at[ridx_v.at[pl.ds(off, _CHUNK)]], ue_v.at[b],
                sem_g[b])
            pltpu.async_copy(
                iemb_hbm.at[cidx_v.at[pl.ds(off, _CHUNK)]], ie_v.at[b],
                sem_g[b])

        def wait_gather(i, b):
            off = pl.multiple_of(i * _CHUNK, 8)
            pltpu.make_async_copy(
                uemb_hbm.at[ridx_v.at[pl.ds(off, _CHUNK)]], ue_v.at[b],
                sem_g[b]).wait()
            pltpu.make_async_copy(
                iemb_hbm.at[cidx_v.at[pl.ds(off, _CHUNK)]], ie_v.at[b],
                sem_g[b]).wait()

        def start_wb(i, b):
            base = pl.multiple_of(wbase + i * _CHUNK, 8)
            pltpu.async_copy(
                ue_v.at[b], ue_out.at[pl.ds(base, _CHUNK)], sem_w[b])
            pltpu.async_copy(
                ie_v.at[b], ie_out.at[pl.ds(base, _CHUNK)], sem_w[b])

        def wait_wb(i, b):
            base = pl.multiple_of(wbase + i * _CHUNK, 8)
            pltpu.make_async_copy(
                ue_v.at[b], ue_out.at[pl.ds(base, _CHUNK)], sem_w[b]).wait()
            pltpu.make_async_copy(
                ie_v.at[b], ie_out.at[pl.ds(base, _CHUNK)], sem_w[b]).wait()

        start_gather(0, 0)
        start_gather(1, 1)

        @pl.loop(0, iters - 3, step=2)
        def _pipe(g):
            wait_gather(g, 0)
            start_wb(g, 0)
            wait_wb(g, 0)
            start_gather(g + 2, 0)
            wait_gather(g + 1, 1)
            start_wb(g + 1, 1)
            wait_wb(g + 1, 1)
            start_gather(g + 3, 1)

        m = iters - 3
        wait_gather(m, 0)
        start_wb(m, 0)
        wait_wb(m, 0)
        start_gather(m + 2, 0)
        wait_gather(m + 1, 1)
        start_wb(m + 1, 1)
        wait_gather(m + 2, 0)
        start_wb(m + 2, 0)
        wait_wb(m + 1, 1)
        wait_wb(m + 2, 0)

    return k(row, col, user_emb, item_emb)


def _tc_mlp(ue8, ie8, w1k, b1t, w2t, b2):
    R = ue8.shape[0]
    BLKR = 4000
    assert R % BLKR == 0
    grid = R // BLKR

    def body(ue_ref, ie_ref, w1_ref, b1_ref, w2_ref, b2_ref, out_ref):
        x = jnp.concatenate(
            [ue_ref[...], ie_ref[...]], axis=1).astype(jnp.bfloat16)
        h = jnp.dot(x, w1_ref[...], preferred_element_type=jnp.float32)
        h = jnp.maximum(h + b1_ref[...], 0.0)
        s = jnp.dot(h.astype(jnp.bfloat16), w2_ref[...],
                    preferred_element_type=jnp.float32) + b2_ref[0, 0]
        out_ref[...] = 1.0 / (1.0 + jnp.exp(-s))

    return pl.pallas_call(
        body,
        grid=(grid,),
        in_specs=[
            pl.BlockSpec((BLKR, 128), lambda i: (i, 0)),
            pl.BlockSpec((BLKR, 128), lambda i: (i, 0)),
            pl.BlockSpec((256, 512), lambda i: (0, 0)),
            pl.BlockSpec((1, 512), lambda i: (0, 0)),
            pl.BlockSpec((512, 8), lambda i: (0, 0)),
            pl.BlockSpec((1, 1), lambda i: (0, 0)),
        ],
        out_specs=pl.BlockSpec((BLKR, 8), lambda i: (i, 0)),
        out_shape=jax.ShapeDtypeStruct((R, 8), jnp.float32),
    )(ue8, ie8, w1k, b1t, w2t, b2)


def kernel(edge_index, edge_weight, user_emb, item_emb,
           W1, b1, W2, b2, W3, b3, g1, be1, g2, be2,
           fcW1, fcb1, fcW2, fcb2):
    E = edge_index.shape[1]
    row = edge_index[0]
    col = edge_index[1]

    eye8 = jnp.eye(8, dtype=jnp.float32)
    w1k = jnp.concatenate(
        [jnp.kron(eye8, fcW1[:16, :]), jnp.kron(eye8, fcW1[16:, :])],
        axis=0).astype(jnp.bfloat16)
    b1t = jnp.tile(fcb1, 8)[None, :]
    w2t = jnp.kron(eye8, fcW2).astype(jnp.bfloat16)
    b2r = fcb2.reshape(1, 1)

    S = 5
    Es = E // S
    preds = []
    for s in range(S):
        ue, ie = _sc_gather(row, col, user_emb, item_emb, s * Es, Es)
        preds.append(
            _tc_mlp(ue.reshape(Es // 8, 128), ie.reshape(Es // 8, 128),
                    w1k, b1t, w2t, b2r))
    pred8 = jnp.concatenate(preds, axis=0)
    return pred8.reshape(E, 1)


# speedup vs baseline: 1.1098x; 1.1098x over previous
import functools

import jax
import jax.numpy as jnp
from jax import lax
from jax.experimental import pallas as pl
from jax.experimental.pallas import tpu as pltpu
from jax.experimental.pallas import tpu_sc as plsc

_NC = 2
_NS = 16
_NW = _NC * _NS
_CHUNK = 1000


def _sc_gather(row, col, user_emb, item_emb, base0, Es):
    D = user_emb.shape[1]
    per_w = Es // _NW
    iters = per_w // _CHUNK
    mesh = plsc.VectorSubcoreMesh(
        core_axis_name="c", subcore_axis_name="s",
        num_cores=_NC, num_subcores=_NS)

    assert iters % 2 == 1 and iters >= 5

    @functools.partial(
        pl.kernel,
        out_type=(jax.ShapeDtypeStruct((Es, D), jnp.float32),
                  jax.ShapeDtypeStruct((Es, D), jnp.float32)),
        mesh=mesh,
        scratch_types=[
            pltpu.VMEM((per_w,), jnp.int32),
            pltpu.VMEM((per_w,), jnp.int32),
            pltpu.VMEM((2, _CHUNK, D), jnp.float32),
            pltpu.VMEM((2, _CHUNK, D), jnp.float32),
            pltpu.SemaphoreType.DMA,
            [pltpu.SemaphoreType.DMA] * 2,
            [pltpu.SemaphoreType.DMA] * 2,
        ],
        compiler_params=pltpu.CompilerParams(use_tc_tiling_on_sc=False),
    )
    def k(row_hbm, col_hbm, uemb_hbm, iemb_hbm, ue_out, ie_out,
          ridx_v, cidx_v, ue_v, ie_v, sem_idx, sem_g, sem_w):
        wid = lax.axis_index("s") * _NC + lax.axis_index("c")
        wbase = pl.multiple_of(wid * per_w, 8)

        cp_r = pltpu.async_copy(
            row_hbm.at[pl.ds(base0 + wbase, per_w)], ridx_v, sem_idx)
        cp_c = pltpu.async_copy(
            col_hbm.at[pl.ds(base0 + wbase, per_w)], cidx_v, sem_idx)
        cp_r.wait()
        cp_c.wait()

        def start_gather(i, b):
            off = pl.multiple_of(i * _CHUNK, 8)
            pltpu.async_copy(
                uemb_hbm.at[ridx_v.at[pl.ds(off, _CHUNK)]], ue_v.at[b],
                sem_g[b])
            pltpu.async_copy(
                iemb_hbm.at[cidx_v.at[pl.ds(off, _CHUNK)]], ie_v.at[b],
                sem_g[b])

        def wait_gather(i, b):
            off = pl.multiple_of(i * _CHUNK, 8)
            pltpu.make_async_copy(
                uemb_hbm.at[ridx_v.at[pl.ds(off, _CHUNK)]], ue_v.at[b],
                sem_g[b]).wait()
            pltpu.make_async_copy(
                iemb_hbm.at[cidx_v.at[pl.ds(off, _CHUNK)]], ie_v.at[b],
                sem_g[b]).wait()

        def start_wb(i, b):
            base = pl.multiple_of(wbase + i * _CHUNK, 8)
            pltpu.async_copy(
                ue_v.at[b], ue_out.at[pl.ds(base, _CHUNK)], sem_w[b])
            pltpu.async_copy(
                ie_v.at[b], ie_out.at[pl.ds(base, _CHUNK)], sem_w[b])

        def wait_wb(i, b):
            base = pl.multiple_of(wbase + i * _CHUNK, 8)
            pltpu.make_async_copy(
                ue_v.at[b], ue_out.at[pl.ds(base, _CHUNK)], sem_w[b]).wait()
            pltpu.make_async_copy(
                ie_v.at[b], ie_out.at[pl.ds(base, _CHUNK)], sem_w[b]).wait()

        start_gather(0, 0)
        start_gather(1, 1)

        @pl.loop(0, iters - 3, step=2)
        def _pipe(g):
            wait_gather(g, 0)
            start_wb(g, 0)
            wait_wb(g, 0)
            start_gather(g + 2, 0)
            wait_gather(g + 1, 1)
            start_wb(g + 1, 1)
            wait_wb(g + 1, 1)
            start_gather(g + 3, 1)

        m = iters - 3
        wait_gather(m, 0)
        start_wb(m, 0)
        wait_wb(m, 0)
        start_gather(m + 2, 0)
        wait_gather(m + 1, 1)
        start_wb(m + 1, 1)
        wait_gather(m + 2, 0)
        start_wb(m + 2, 0)
        wait_wb(m + 1, 1)
        wait_wb(m + 2, 0)

    return k(row, col, user_emb, item_emb)


def _tc_mlp(ue8, ie8, w1k, b1t, w2t, b2):
    R = ue8.shape[0]
    BLKR = 4000
    assert R % BLKR == 0
    grid = R // BLKR

    def body(ue_ref, ie_ref, w1_ref, b1_ref, w2_ref, b2_ref, out_ref):
        x = jnp.concatenate(
            [ue_ref[...], ie_ref[...]], axis=1).astype(jnp.bfloat16)
        h = jnp.dot(x, w1_ref[...], preferred_element_type=jnp.float32)
        h = jnp.maximum(h + b1_ref[...], 0.0)
        s = jnp.dot(h.astype(jnp.bfloat16), w2_ref[...],
                    preferred_element_type=jnp.float32) + b2_ref[0, 0]
        out_ref[...] = 1.0 / (1.0 + jnp.exp(-s))

    return pl.pallas_call(
        body,
        grid=(grid,),
        in_specs=[
            pl.BlockSpec((BLKR, 128), lambda i: (i, 0)),
            pl.BlockSpec((BLKR, 128), lambda i: (i, 0)),
            pl.BlockSpec((256, 512), lambda i: (0, 0)),
            pl.BlockSpec((1, 512), lambda i: (0, 0)),
            pl.BlockSpec((512, 8), lambda i: (0, 0)),
            pl.BlockSpec((1, 1), lambda i: (0, 0)),
        ],
        out_specs=pl.BlockSpec((BLKR, 8), lambda i: (i, 0)),
        out_shape=jax.ShapeDtypeStruct((R, 8), jnp.float32),
    )(ue8, ie8, w1k, b1t, w2t, b2)


def kernel(edge_index, edge_weight, user_emb, item_emb,
           W1, b1, W2, b2, W3, b3, g1, be1, g2, be2,
           fcW1, fcb1, fcW2, fcb2):
    E = edge_index.shape[1]
    row = edge_index[0]
    col = edge_index[1]

    eye8 = jnp.eye(8, dtype=jnp.float32)
    w1k = jnp.concatenate(
        [jnp.kron(eye8, fcW1[:16, :]), jnp.kron(eye8, fcW1[16:, :])],
        axis=0).astype(jnp.bfloat16)
    b1t = jnp.tile(fcb1, 8)[None, :]
    w2t = jnp.kron(eye8, fcW2).astype(jnp.bfloat16)
    b2r = fcb2.reshape(1, 1)

    S = 1
    Es = E // S
    preds = []
    for s in range(S):
        ue, ie = _sc_gather(row, col, user_emb, item_emb, s * Es, Es)
        preds.append(
            _tc_mlp(ue.reshape(Es // 8, 128), ie.reshape(Es // 8, 128),
                    w1k, b1t, w2t, b2r))
    pred8 = jnp.concatenate(preds, axis=0)
    return pred8.reshape(E, 1)
